# Initial kernel scaffold; baseline (speedup 1.0000x reference)
import jax
import jax.numpy as jnp
from jax.experimental import pallas as pl


def kernel(atomic_number, coordinate, edge_index, batch, embed_table, W_rbf0, b_rbf0, W_up0, b_up0, W_rbf1, b_rbf1, W_up1, b_up1, W_out, b_out):
    raise NotImplementedError("write your pallas kernel here")


# speedup vs baseline: 3.1389x; 3.1389x over previous
import functools

import jax
import jax.numpy as jnp
from jax import lax
from jax.experimental import pallas as pl
from jax.experimental.pallas import tpu as pltpu
from jax.experimental.pallas import tpu_sc as plsc

N_NODES = 10000
N_EDGES = 320000
C = 128
NB = 32
N_SPECIES = 100
G = 64
CUTOFF = 5.0
GAMMA = (NB / CUTOFF) ** 2

NCORE = 2
NSUB = 16
NW = NCORE * NSUB
NPAD = 10240
ROWS_PER_W = NPAD // NW
E_PER_W = N_EDGES // NW
IW = 80
JROWS = E_PER_W // IW
JPAD = 128
ECH = 2000
EB = 512
NBLK = 256

_mesh = plsc.VectorSubcoreMesh(
    core_axis_name="c", subcore_axis_name="s",
    num_cores=NCORE, num_subcores=NSUB)
_sc_params = pltpu.CompilerParams(needs_layout_passes=False)


@functools.partial(
    pl.kernel,
    out_type=[jax.ShapeDtypeStruct((NPAD, C), jnp.float32),
              jax.ShapeDtypeStruct((N_EDGES,), jnp.float32)],
    mesh=_mesh,
    compiler_params=_sc_params,
    scratch_types=[
        pltpu.VMEM((NPAD,), jnp.float32),
        pltpu.VMEM((NPAD,), jnp.float32),
        pltpu.VMEM((NPAD,), jnp.float32),
        pltpu.VMEM((ECH,), jnp.int32),
        pltpu.VMEM((ECH,), jnp.int32),
        pltpu.VMEM((ECH,), jnp.float32),
        pltpu.VMEM((4, IW), jnp.int32),
        pltpu.VMEM((IW, C), jnp.float32),
        pltpu.SemaphoreType.DMA,
    ],
)
def _sc_prep(an2d_h, cx_h, cy_h, cz_h, src_h, dst_h, embed_h, x0_h, sq_h,
             cxv, cyv, czv, sidx, didx, sqv, anv, xrows, sem):
    cid = lax.axis_index("c")
    sid = lax.axis_index("s")
    wid = cid * NSUB + sid

    pltpu.sync_copy(an2d_h.at[pl.ds(wid * 4, 4)], anv)
    for j in range(4):
        pltpu.async_copy(embed_h.at[anv.at[j]], xrows, sem).wait()
        pltpu.sync_copy(xrows, x0_h.at[pl.ds(wid * ROWS_PER_W + j * IW, IW)])

    pltpu.sync_copy(cx_h, cxv)
    pltpu.sync_copy(cy_h, cyv)
    pltpu.sync_copy(cz_h, czv)

    ebase = wid * E_PER_W

    def chunk_body(k, _):
        base = ebase + k * ECH
        pltpu.sync_copy(src_h.at[pl.ds(base, ECH)], sidx)
        pltpu.sync_copy(dst_h.at[pl.ds(base, ECH)], didx)

        def g_body(g, _):
            sl = pl.ds(g * 16, 16)
            s16 = sidx[sl]
            d16 = didx[sl]
            dx = plsc.load_gather(cxv, [d16]) - plsc.load_gather(cxv, [s16])
            dy = plsc.load_gather(cyv, [d16]) - plsc.load_gather(cyv, [s16])
            dz = plsc.load_gather(czv, [d16]) - plsc.load_gather(czv, [s16])
            sqv[sl] = dx * dx + dy * dy + dz * dz
            return 0

        lax.fori_loop(0, ECH // 16, g_body, 0)
        pltpu.sync_copy(sqv, sq_h.at[pl.ds(base, ECH)])
        return 0

    lax.fori_loop(0, E_PER_W // ECH, chunk_body, 0)


@functools.partial(
    pl.kernel,
    out_type=jax.ShapeDtypeStruct((NCORE * NPAD, C), jnp.float32),
    mesh=_mesh,
    compiler_params=_sc_params,
    scratch_types=[
        pltpu.VMEM((32, IW), jnp.int32),
        pltpu.VMEM((32, IW), jnp.int32),
        pltpu.VMEM((IW, C), jnp.float32),
        pltpu.VMEM((IW, C), jnp.float32),
        pltpu.VMEM_SHARED((NPAD, C), jnp.float32),
        pltpu.SemaphoreType.DMA,
    ],
)
def _sc_msgpass(x_h, filt_h, src2_h, dst2_h, parts_h,
                sidx, didx, rows, fv, acc, sem):
    cid = lax.axis_index("c")
    sid = lax.axis_index("s")
    wid = cid * NSUB + sid
    zrows = NPAD // NSUB

    def zr(r, _):
        for c8 in range(C // 16):
            fv[r, pl.ds(c8 * 16, 16)] = jnp.zeros((16,), jnp.float32)
        return 0
    lax.fori_loop(0, IW, zr, 0)

    def zc(i, _):
        pltpu.sync_copy(fv, acc.at[pl.ds(sid * zrows + i * IW, IW)])
        return 0
    lax.fori_loop(0, zrows // IW, zc, 0)
    plsc.subcore_barrier()

    ebase = wid * E_PER_W

    def cb(c, _):
        pltpu.sync_copy(src2_h.at[pl.ds(wid * JPAD + c * 32, 32)], sidx)
        pltpu.sync_copy(dst2_h.at[pl.ds(wid * JPAD + c * 32, 32)], didx)
        nrows = jnp.minimum(32, JROWS - c * 32)

        def jb(jj, _):
            j = c * 32 + jj
            pltpu.async_copy(x_h.at[sidx.at[jj]], rows, sem).wait()
            pltpu.sync_copy(filt_h.at[pl.ds(ebase + j * IW, IW)], fv)

            def mb(r, _):
                for c8 in range(C // 16):
                    sl = pl.ds(c8 * 16, 16)
                    fv[r, sl] = fv[r, sl] * rows[r, sl]
                return 0
            lax.fori_loop(0, IW, mb, 0)

            pltpu.sync_copy(fv, acc.at[didx.at[jj]], add=True)
            return 0

        lax.fori_loop(0, nrows, jb, 0)
        return 0

    lax.fori_loop(0, (JROWS + 31) // 32, cb, 0)
    plsc.subcore_barrier()

    pltpu.sync_copy(acc.at[pl.ds(sid * zrows, zrows)],
                    parts_h.at[pl.ds(cid * NPAD + sid * zrows, zrows)])


def _sigmoid(z):
    return 1.0 / (1.0 + jnp.exp(-z))


def _filters_body(sq_ref, w0_ref, b0_ref, w1_ref, b1_ref, f0_ref, f1_ref):
    d = jnp.sqrt(sq_ref[:] + 1e-8)
    cent = lax.broadcasted_iota(jnp.int32, (1, NB), 1).astype(jnp.float32)
    cent = cent * (CUTOFF / (NB - 1))
    diff = d - cent
    rbf = jnp.exp(-GAMMA * diff * diff)
    z0 = jnp.dot(rbf, w0_ref[:], preferred_element_type=jnp.float32) + b0_ref[:]
    f0_ref[:] = z0 * _sigmoid(z0)
    z1 = jnp.dot(rbf, w1_ref[:], preferred_element_type=jnp.float32) + b1_ref[:]
    f1_ref[:] = z1 * _sigmoid(z1)


def _tc_filters(sq2, w0, b0, w1, b1):
    grid = N_EDGES // EB
    return pl.pallas_call(
        _filters_body,
        grid=(grid,),
        in_specs=[
            pl.BlockSpec((EB, 1), lambda i: (i, 0)),
            pl.BlockSpec((NB, C), lambda i: (0, 0)),
            pl.BlockSpec((1, C), lambda i: (0, 0)),
            pl.BlockSpec((NB, C), lambda i: (0, 0)),
            pl.BlockSpec((1, C), lambda i: (0, 0)),
        ],
        out_specs=[
            pl.BlockSpec((EB, C), lambda i: (i, 0)),
            pl.BlockSpec((EB, C), lambda i: (i, 0)),
        ],
        out_shape=[
            jax.ShapeDtypeStruct((N_EDGES, C), jnp.float32),
            jax.ShapeDtypeStruct((N_EDGES, C), jnp.float32),
        ],
    )(sq2, w0, b0, w1, b1)


def _update_body(x_ref, p_ref, w_ref, b_ref, o_ref):
    h = x_ref[:] + p_ref[0] + p_ref[1]
    z = jnp.dot(h, w_ref[:], preferred_element_type=jnp.float32) + b_ref[:]
    o_ref[:] = z * _sigmoid(z)


def _tc_update(x, parts, w, b):
    grid = NPAD // NBLK
    return pl.pallas_call(
        _update_body,
        grid=(grid,),
        in_specs=[
            pl.BlockSpec((NBLK, C), lambda i: (i, 0)),
            pl.BlockSpec((NCORE, NBLK, C), lambda i: (0, i, 0)),
            pl.BlockSpec((C, C), lambda i: (0, 0)),
            pl.BlockSpec((1, C), lambda i: (0, 0)),
        ],
        out_specs=pl.BlockSpec((NBLK, C), lambda i: (i, 0)),
        out_shape=jax.ShapeDtypeStruct((NPAD, C), jnp.float32),
    )(x, parts, w, b)


def _final_body(x_ref, p_ref, w_ref, b_ref, bat_ref, wo_ref, bo_ref,
                gx_ref, gc_ref, go_ref):
    i = pl.program_id(0)
    h = x_ref[:] + p_ref[0] + p_ref[1]
    z = jnp.dot(h, w_ref[:], preferred_element_type=jnp.float32) + b_ref[:]
    x2 = z * _sigmoid(z)
    gids = lax.broadcasted_iota(jnp.int32, (1, G), 1)
    onehot = (bat_ref[:] == gids).astype(jnp.float32)
    dn = (((0,), (0,)), ((), ()))
    gpart = lax.dot_general(onehot, x2, dn, preferred_element_type=jnp.float32)
    cpart = lax.dot_general(onehot, jnp.ones_like(x2), dn,
                            preferred_element_type=jnp.float32)

    @pl.when(i == 0)
    def _():
        gx_ref[:] = gpart
        gc_ref[:] = cpart

    @pl.when(i > 0)
    def _():
        gx_ref[:] = gx_ref[:] + gpart
        gc_ref[:] = gc_ref[:] + cpart

    @pl.when(i == pl.num_programs(0) - 1)
    def _():
        go_ref[:] = (jnp.dot(gx_ref[:], wo_ref[:],
                             preferred_element_type=jnp.float32)
                     + gc_ref[:, 0:1] * bo_ref[0, 0])


def _tc_final(x, parts, w, b, bat2, wo, bo):
    grid = NPAD // NBLK
    _, _, go = pl.pallas_call(
        _final_body,
        grid=(grid,),
        in_specs=[
            pl.BlockSpec((NBLK, C), lambda i: (i, 0)),
            pl.BlockSpec((NCORE, NBLK, C), lambda i: (0, i, 0)),
            pl.BlockSpec((C, C), lambda i: (0, 0)),
            pl.BlockSpec((1, C), lambda i: (0, 0)),
            pl.BlockSpec((NBLK, 1), lambda i: (i, 0)),
            pl.BlockSpec((C, 1), lambda i: (0, 0)),
            pl.BlockSpec((1, 1), lambda i: (0, 0)),
        ],
        out_specs=[
            pl.BlockSpec((G, C), lambda i: (0, 0)),
            pl.BlockSpec((G, C), lambda i: (0, 0)),
            pl.BlockSpec((G, 1), lambda i: (0, 0)),
        ],
        out_shape=[
            jax.ShapeDtypeStruct((G, C), jnp.float32),
            jax.ShapeDtypeStruct((G, C), jnp.float32),
            jax.ShapeDtypeStruct((G, 1), jnp.float32),
        ],
        compiler_params=pltpu.CompilerParams(
            dimension_semantics=("arbitrary",)),
    )(x, parts, w, b, bat2, wo, bo)
    return go


def kernel(atomic_number, coordinate, edge_index, batch, embed_table,
           W_rbf0, b_rbf0, W_up0, b_up0,
           W_rbf1, b_rbf1, W_up1, b_up1,
           W_out, b_out):
    an = jnp.pad(atomic_number.astype(jnp.int32), (0, NPAD - N_NODES))
    an2d = an.reshape(NPAD // IW, IW)
    coord_t = jnp.pad(coordinate.T.astype(jnp.float32),
                      ((0, 0), (0, NPAD - N_NODES)))
    cx, cy, cz = coord_t[0], coord_t[1], coord_t[2]
    src = edge_index[0].astype(jnp.int32)
    dst = edge_index[1].astype(jnp.int32)
    src2 = jnp.pad(src.reshape(NW, JROWS, IW),
                   ((0, 0), (0, JPAD - JROWS), (0, 0))).reshape(NW * JPAD, IW)
    dst2 = jnp.pad(dst.reshape(NW, JROWS, IW),
                   ((0, 0), (0, JPAD - JROWS), (0, 0))).reshape(NW * JPAD, IW)
    bat2 = jnp.pad(batch.astype(jnp.int32), (0, NPAD - N_NODES),
                   constant_values=G).reshape(NPAD, 1)

    x0, sq = _sc_prep(an2d, cx, cy, cz, src, dst, embed_table)
    filt0, filt1 = _tc_filters(sq.reshape(N_EDGES, 1),
                               W_rbf0, b_rbf0.reshape(1, C),
                               W_rbf1, b_rbf1.reshape(1, C))
    parts0 = _sc_msgpass(x0, filt0, src2, dst2).reshape(NCORE, NPAD, C)
    x1 = _tc_update(x0, parts0, W_up0, b_up0.reshape(1, C))
    parts1 = _sc_msgpass(x1, filt1, src2, dst2).reshape(NCORE, NPAD, C)
    go = _tc_final(x1, parts1, W_up1, b_up1.reshape(1, C),
                   bat2, W_out, b_out.reshape(1, 1))
    return go
